# baseline (speedup 1.0000x reference)
import functools

import jax
import jax.numpy as jnp
from jax import lax
from jax.experimental import pallas as pl
from jax.experimental.pallas import tpu as pltpu
from jax.experimental.pallas import tpu_sc as plsc

VOCAB = 1000000
EMBED_DIM = 64
BATCH = 4096
HIST = 200
HIDDEN = 512
OUT = 128

NC = 2
NS = 16
NW = NC * NS
ROWS_PER_W = BATCH // NW
IDX_SPLIT = 2
IDX_W = HIST // IDX_SPLIT
NBUF = 2


def _sc_pool(x2, table):
    mesh = plsc.VectorSubcoreMesh(core_axis_name="c", subcore_axis_name="s")

    @functools.partial(
        pl.kernel,
        out_type=jax.ShapeDtypeStruct((BATCH, EMBED_DIM), jnp.float32),
        mesh=mesh,
        scratch_types=[
            pltpu.VMEM((ROWS_PER_W * IDX_SPLIT, IDX_W), jnp.int32),
            pltpu.VMEM((NBUF, HIST, EMBED_DIM), jnp.float32),
            pltpu.VMEM((ROWS_PER_W, EMBED_DIM), jnp.float32),
            pltpu.SemaphoreType.DMA,
            pltpu.SemaphoreType.DMA,
        ],
        compiler_params=pltpu.CompilerParams(use_tc_tiling_on_sc=False),
    )
    def pool(x_hbm, table_hbm, out_hbm, idx_v, gbuf, outbuf, sem0, sem1):
        wid = lax.axis_index("s") * NC + lax.axis_index("c")
        pltpu.sync_copy(
            x_hbm.at[pl.ds(wid * ROWS_PER_W * IDX_SPLIT, ROWS_PER_W * IDX_SPLIT)],
            idx_v,
        )
        sems = (sem0, sem1)

        def issue(b, slot, sem):
            pltpu.async_copy(
                table_hbm.at[idx_v.at[IDX_SPLIT * b]],
                gbuf.at[slot, pl.ds(0, IDX_W)], sem)
            pltpu.async_copy(
                table_hbm.at[idx_v.at[IDX_SPLIT * b + 1]],
                gbuf.at[slot, pl.ds(IDX_W, IDX_W)], sem)

        for s in range(NBUF):
            issue(s, s, sems[s])

        @pl.loop(0, ROWS_PER_W, step=NBUF)
        def body(b):
            for s in range(NBUF):
                bb = b + s
                pltpu.make_async_copy(
                    table_hbm.at[pl.ds(0, HIST)], gbuf.at[s], sems[s]).wait()

                zero = jnp.zeros((16,), jnp.float32)

                def red(r, carry, s=s):
                    a0, a1, a2, a3 = carry
                    a0 = a0 + gbuf[s, r, 0:16]
                    a1 = a1 + gbuf[s, r, 16:32]
                    a2 = a2 + gbuf[s, r, 32:48]
                    a3 = a3 + gbuf[s, r, 48:64]
                    return (a0, a1, a2, a3)

                a0, a1, a2, a3 = pl.loop(
                    0, HIST, init_carry=(zero, zero, zero, zero), unroll=8)(red)
                outbuf[bb, 0:16] = a0
                outbuf[bb, 16:32] = a1
                outbuf[bb, 32:48] = a2
                outbuf[bb, 48:64] = a3

                @pl.when(bb + NBUF < ROWS_PER_W)
                def _(bb=bb, s=s):
                    issue(bb + NBUF, s, sems[s])

        pltpu.sync_copy(outbuf, out_hbm.at[pl.ds(wid * ROWS_PER_W, ROWS_PER_W)])

    return pool(x2, table)


BLK = 512


def _mlp(x, sums, W1, b1, W2, b2):
    def mlp_body(x_ref, s_ref, w1_ref, b1_ref, w2_ref, b2_ref, o_ref):
        xi = x_ref[...]
        lengths = jnp.sum((xi != 0).astype(jnp.float32), axis=1, keepdims=True)
        pooled = s_ref[...] / lengths
        h = lax.dot_general(
            pooled, w1_ref[...], (((1,), (1,)), ((), ())),
            precision=lax.Precision.HIGHEST,
            preferred_element_type=jnp.float32) + b1_ref[...]
        h = jnp.maximum(h, 0.0)
        o_ref[...] = lax.dot_general(
            h, w2_ref[...], (((1,), (1,)), ((), ())),
            precision=lax.Precision.HIGHEST,
            preferred_element_type=jnp.float32) + b2_ref[...]

    return pl.pallas_call(
        mlp_body,
        grid=(BATCH // BLK,),
        in_specs=[
            pl.BlockSpec((BLK, HIST), lambda i: (i, 0)),
            pl.BlockSpec((BLK, EMBED_DIM), lambda i: (i, 0)),
            pl.BlockSpec((HIDDEN, EMBED_DIM), lambda i: (0, 0)),
            pl.BlockSpec((1, HIDDEN), lambda i: (0, 0)),
            pl.BlockSpec((OUT, HIDDEN), lambda i: (0, 0)),
            pl.BlockSpec((1, OUT), lambda i: (0, 0)),
        ],
        out_specs=pl.BlockSpec((BLK, OUT), lambda i: (i, 0)),
        out_shape=jax.ShapeDtypeStruct((BATCH, OUT), jnp.float32),
    )(x, sums, W1, b1.reshape(1, HIDDEN), W2, b2.reshape(1, OUT))


def kernel(x, table, W1, b1, W2, b2):
    x2 = x.reshape(BATCH * IDX_SPLIT, IDX_W)
    sums = _sc_pool(x2, table)
    return _mlp(x, sums, W1, b1, W2, b2)


# speedup vs baseline: 1.0559x; 1.0559x over previous
import functools

import jax
import jax.numpy as jnp
from jax import lax
from jax.experimental import pallas as pl
from jax.experimental.pallas import tpu as pltpu
from jax.experimental.pallas import tpu_sc as plsc

VOCAB = 1000000
EMBED_DIM = 64
BATCH = 4096
HIST = 200
HIDDEN = 512
OUT = 128

NC = 2
NS = 16
NW = NC * NS
ROWS_PER_W = BATCH // NW
IDX_SPLIT = 2
IDX_W = HIST // IDX_SPLIT
NBUF = 4


def _sc_pool(x2, table):
    mesh = plsc.VectorSubcoreMesh(core_axis_name="c", subcore_axis_name="s")

    @functools.partial(
        pl.kernel,
        out_type=jax.ShapeDtypeStruct((BATCH, EMBED_DIM), jnp.float32),
        mesh=mesh,
        scratch_types=[
            pltpu.VMEM((ROWS_PER_W * IDX_SPLIT, IDX_W), jnp.int32),
            pltpu.VMEM((NBUF, IDX_SPLIT, IDX_W, EMBED_DIM), jnp.float32),
            pltpu.VMEM((ROWS_PER_W, EMBED_DIM), jnp.float32),
            pltpu.SemaphoreType.DMA,
            pltpu.SemaphoreType.DMA,
            pltpu.SemaphoreType.DMA,
            pltpu.SemaphoreType.DMA,
        ],
        compiler_params=pltpu.CompilerParams(use_tc_tiling_on_sc=False),
    )
    def pool(x_hbm, table_hbm, out_hbm, idx_v, gbuf, outbuf, sem0, sem1, sem2, sem3):
        wid = lax.axis_index("s") * NC + lax.axis_index("c")
        pltpu.sync_copy(
            x_hbm.at[pl.ds(wid * ROWS_PER_W * IDX_SPLIT, ROWS_PER_W * IDX_SPLIT)],
            idx_v,
        )
        sems = (sem0, sem1, sem2, sem3)

        def issue(b, slot, sem):
            for h in range(IDX_SPLIT):
                pltpu.async_copy(
                    table_hbm.at[idx_v.at[IDX_SPLIT * b + h]],
                    gbuf.at[slot, h], sem)

        for s in range(NBUF):
            issue(s, s, sems[s])

        @pl.loop(0, ROWS_PER_W, step=NBUF)
        def body(b):
            for s in range(NBUF):
                bb = b + s
                for h in range(IDX_SPLIT):
                    pltpu.make_async_copy(
                        table_hbm.at[idx_v.at[IDX_SPLIT * bb + h]],
                        gbuf.at[s, h], sems[s]).wait()

                zero = jnp.zeros((16,), jnp.float32)

                def red(r, carry, s=s):
                    a0, a1, a2, a3 = carry
                    for h in range(IDX_SPLIT):
                        a0 = a0 + gbuf[s, h, r, 0:16]
                        a1 = a1 + gbuf[s, h, r, 16:32]
                        a2 = a2 + gbuf[s, h, r, 32:48]
                        a3 = a3 + gbuf[s, h, r, 48:64]
                    return (a0, a1, a2, a3)

                a0, a1, a2, a3 = pl.loop(
                    0, IDX_W, init_carry=(zero, zero, zero, zero), unroll=4)(red)
                outbuf[bb, 0:16] = a0
                outbuf[bb, 16:32] = a1
                outbuf[bb, 32:48] = a2
                outbuf[bb, 48:64] = a3

                @pl.when(bb + NBUF < ROWS_PER_W)
                def _(bb=bb, s=s):
                    issue(bb + NBUF, s, sems[s])

        pltpu.sync_copy(outbuf, out_hbm.at[pl.ds(wid * ROWS_PER_W, ROWS_PER_W)])

    return pool(x2, table)


BLK = 512


def _mlp(x, sums, W1, b1, W2, b2):
    def mlp_body(x_ref, s_ref, w1_ref, b1_ref, w2_ref, b2_ref, o_ref):
        xi = x_ref[...]
        lengths = jnp.sum((xi != 0).astype(jnp.float32), axis=1, keepdims=True)
        pooled = s_ref[...] / lengths
        h = lax.dot_general(
            pooled, w1_ref[...], (((1,), (1,)), ((), ())),
            precision=lax.Precision.HIGHEST,
            preferred_element_type=jnp.float32) + b1_ref[...]
        h = jnp.maximum(h, 0.0)
        o_ref[...] = lax.dot_general(
            h, w2_ref[...], (((1,), (1,)), ((), ())),
            precision=lax.Precision.HIGHEST,
            preferred_element_type=jnp.float32) + b2_ref[...]

    return pl.pallas_call(
        mlp_body,
        grid=(BATCH // BLK,),
        in_specs=[
            pl.BlockSpec((BLK, HIST), lambda i: (i, 0)),
            pl.BlockSpec((BLK, EMBED_DIM), lambda i: (i, 0)),
            pl.BlockSpec((HIDDEN, EMBED_DIM), lambda i: (0, 0)),
            pl.BlockSpec((1, HIDDEN), lambda i: (0, 0)),
            pl.BlockSpec((OUT, HIDDEN), lambda i: (0, 0)),
            pl.BlockSpec((1, OUT), lambda i: (0, 0)),
        ],
        out_specs=pl.BlockSpec((BLK, OUT), lambda i: (i, 0)),
        out_shape=jax.ShapeDtypeStruct((BATCH, OUT), jnp.float32),
    )(x, sums, W1, b1.reshape(1, HIDDEN), W2, b2.reshape(1, OUT))


def kernel(x, table, W1, b1, W2, b2):
    x2 = x.reshape(BATCH * IDX_SPLIT, IDX_W)
    sums = _sc_pool(x2, table)
    return _mlp(x, sums, W1, b1, W2, b2)


# speedup vs baseline: 1.0625x; 1.0063x over previous
import functools

import jax
import jax.numpy as jnp
from jax import lax
from jax.experimental import pallas as pl
from jax.experimental.pallas import tpu as pltpu
from jax.experimental.pallas import tpu_sc as plsc

VOCAB = 1000000
EMBED_DIM = 64
BATCH = 4096
HIST = 200
HIDDEN = 512
OUT = 128

NC = 2
NS = 16
NW = NC * NS
ROWS_PER_W = BATCH // NW
IDX_SPLIT = 1
IDX_W = HIST // IDX_SPLIT
NBUF = 4


def _sc_pool(x2, table):
    mesh = plsc.VectorSubcoreMesh(core_axis_name="c", subcore_axis_name="s")

    @functools.partial(
        pl.kernel,
        out_type=jax.ShapeDtypeStruct((BATCH, EMBED_DIM), jnp.float32),
        mesh=mesh,
        scratch_types=[
            pltpu.VMEM((ROWS_PER_W * IDX_SPLIT, IDX_W), jnp.int32),
            pltpu.VMEM((NBUF, IDX_SPLIT, IDX_W, EMBED_DIM), jnp.float32),
            pltpu.VMEM((ROWS_PER_W, EMBED_DIM), jnp.float32),
            pltpu.SemaphoreType.DMA,
            pltpu.SemaphoreType.DMA,
            pltpu.SemaphoreType.DMA,
            pltpu.SemaphoreType.DMA,
        ],
        compiler_params=pltpu.CompilerParams(use_tc_tiling_on_sc=False),
    )
    def pool(x_hbm, table_hbm, out_hbm, idx_v, gbuf, outbuf, sem0, sem1, sem2, sem3):
        wid = lax.axis_index("s") * NC + lax.axis_index("c")
        pltpu.sync_copy(
            x_hbm.at[pl.ds(wid * ROWS_PER_W * IDX_SPLIT, ROWS_PER_W * IDX_SPLIT)],
            idx_v,
        )
        sems = (sem0, sem1, sem2, sem3)

        def issue(b, slot, sem):
            for h in range(IDX_SPLIT):
                pltpu.async_copy(
                    table_hbm.at[idx_v.at[IDX_SPLIT * b + h]],
                    gbuf.at[slot, h], sem)

        for s in range(NBUF):
            issue(s, s, sems[s])

        @pl.loop(0, ROWS_PER_W, step=NBUF)
        def body(b):
            for s in range(NBUF):
                bb = b + s
                for h in range(IDX_SPLIT):
                    pltpu.make_async_copy(
                        table_hbm.at[idx_v.at[IDX_SPLIT * bb + h]],
                        gbuf.at[s, h], sems[s]).wait()

                zero = jnp.zeros((16,), jnp.float32)

                def red(r, carry, s=s):
                    a0, a1, a2, a3 = carry
                    for h in range(IDX_SPLIT):
                        a0 = a0 + gbuf[s, h, r, 0:16]
                        a1 = a1 + gbuf[s, h, r, 16:32]
                        a2 = a2 + gbuf[s, h, r, 32:48]
                        a3 = a3 + gbuf[s, h, r, 48:64]
                    return (a0, a1, a2, a3)

                a0, a1, a2, a3 = pl.loop(
                    0, IDX_W, init_carry=(zero, zero, zero, zero), unroll=4)(red)
                outbuf[bb, 0:16] = a0
                outbuf[bb, 16:32] = a1
                outbuf[bb, 32:48] = a2
                outbuf[bb, 48:64] = a3

                @pl.when(bb + NBUF < ROWS_PER_W)
                def _(bb=bb, s=s):
                    issue(bb + NBUF, s, sems[s])

        pltpu.sync_copy(outbuf, out_hbm.at[pl.ds(wid * ROWS_PER_W, ROWS_PER_W)])

    return pool(x2, table)


BLK = 512


def _mlp(x, sums, W1, b1, W2, b2):
    def mlp_body(x_ref, s_ref, w1_ref, b1_ref, w2_ref, b2_ref, o_ref):
        xi = x_ref[...]
        lengths = jnp.sum((xi != 0).astype(jnp.float32), axis=1, keepdims=True)
        pooled = s_ref[...] / lengths
        h = lax.dot_general(
            pooled, w1_ref[...], (((1,), (1,)), ((), ())),
            precision=lax.Precision.HIGHEST,
            preferred_element_type=jnp.float32) + b1_ref[...]
        h = jnp.maximum(h, 0.0)
        o_ref[...] = lax.dot_general(
            h, w2_ref[...], (((1,), (1,)), ((), ())),
            precision=lax.Precision.HIGHEST,
            preferred_element_type=jnp.float32) + b2_ref[...]

    return pl.pallas_call(
        mlp_body,
        grid=(BATCH // BLK,),
        in_specs=[
            pl.BlockSpec((BLK, HIST), lambda i: (i, 0)),
            pl.BlockSpec((BLK, EMBED_DIM), lambda i: (i, 0)),
            pl.BlockSpec((HIDDEN, EMBED_DIM), lambda i: (0, 0)),
            pl.BlockSpec((1, HIDDEN), lambda i: (0, 0)),
            pl.BlockSpec((OUT, HIDDEN), lambda i: (0, 0)),
            pl.BlockSpec((1, OUT), lambda i: (0, 0)),
        ],
        out_specs=pl.BlockSpec((BLK, OUT), lambda i: (i, 0)),
        out_shape=jax.ShapeDtypeStruct((BATCH, OUT), jnp.float32),
    )(x, sums, W1, b1.reshape(1, HIDDEN), W2, b2.reshape(1, OUT))


def kernel(x, table, W1, b1, W2, b2):
    x2 = x.reshape(BATCH * IDX_SPLIT, IDX_W)
    sums = _sc_pool(x2, table)
    return _mlp(x, sums, W1, b1, W2, b2)
